# baseline (speedup 1.0000x reference)
import functools

import jax
import jax.numpy as jnp
from jax import lax
from jax.experimental import pallas as pl
from jax.experimental.pallas import tpu as pltpu
from jax.experimental.pallas import tpu_sc as plsc

N, D, K = 8192, 256, 8192
BN, BK = 512, 4096


def _dist_body(x_ref, c_ref, x2_ref, c2_ref, d_ref, lbl_ref, gmin_ref, garg_ref):
    j = pl.program_id(1)
    nj = pl.num_programs(1)
    x = x_ref[...]
    c = c_ref[...]

    mm = lax.dot_general(
        x, c, dimension_numbers=(((1,), (1,)), ((), ())),
        preferred_element_type=jnp.float32,
    )
    d = (x2_ref[...] + c2_ref[...]) - 2.0 * mm
    d_ref[...] = d

    lmin = jnp.min(d, axis=1, keepdims=True)
    col = lax.broadcasted_iota(jnp.int32, (1, BK), 1).astype(jnp.float32)
    larg = jnp.min(jnp.where(d == lmin, col, 2.0 * K), axis=1, keepdims=True)
    larg = larg + jnp.float32(BK) * j.astype(jnp.float32)

    @pl.when(j == 0)
    def _():
        gmin_ref[...] = lmin
        garg_ref[...] = larg

    @pl.when(j > 0)
    def _():
        better = lmin < gmin_ref[...]
        gmin_ref[...] = jnp.where(better, lmin, gmin_ref[...])
        garg_ref[...] = jnp.where(better, larg, garg_ref[...])

    @pl.when(j == nj - 1)
    def _():
        lbl_ref[...] = garg_ref[...].astype(jnp.int32)


def _distances_and_labels(x, c, x2, c2):
    return pl.pallas_call(
        _dist_body,
        grid=(N // BN, K // BK),
        in_specs=[
            pl.BlockSpec((BN, D), lambda i, j: (i, 0)),
            pl.BlockSpec((BK, D), lambda i, j: (j, 0)),
            pl.BlockSpec((BN, 1), lambda i, j: (i, 0)),
            pl.BlockSpec((1, BK), lambda i, j: (0, j)),
        ],
        out_specs=[
            pl.BlockSpec((BN, BK), lambda i, j: (i, j)),
            pl.BlockSpec((BN, 1), lambda i, j: (i, 0)),
        ],
        out_shape=[
            jax.ShapeDtypeStruct((N, K), jnp.float32),
            jax.ShapeDtypeStruct((N, 1), jnp.int32),
        ],
        scratch_shapes=[
            pltpu.VMEM((BN, 1), jnp.float32),
            pltpu.VMEM((BN, 1), jnp.float32),
        ],
        compiler_params=pltpu.CompilerParams(
            dimension_semantics=("arbitrary", "arbitrary"),
        ),
    )(x, c, x2, c2)


def _gather_preds(codebook, labels):
    info = plsc.get_sparse_core_info()
    nw = info.num_cores * info.num_subcores
    b_per_w = N // nw
    mesh = plsc.VectorSubcoreMesh(core_axis_name="c", subcore_axis_name="s")

    half = b_per_w // 2

    @functools.partial(
        pl.kernel, mesh=mesh,
        out_type=jax.ShapeDtypeStruct((N, D), jnp.float32),
        scratch_types=[
            pltpu.VMEM((b_per_w,), jnp.int32),
            pltpu.VMEM((half, D), jnp.float32),
            pltpu.VMEM((half, D), jnp.float32),
            pltpu.SemaphoreType.DMA,
            pltpu.SemaphoreType.DMA,
            pltpu.SemaphoreType.DMA,
            pltpu.SemaphoreType.DMA,
        ],
    )
    def k(table_hbm, idx_hbm, out_hbm, idx_v, rows0_v, rows1_v,
          gsem0, gsem1, osem0, osem1):
        wid = lax.axis_index("s") * info.num_cores + lax.axis_index("c")
        base = wid * b_per_w
        pltpu.sync_copy(idx_hbm.at[pl.ds(base, b_per_w)], idx_v)
        g0 = pltpu.async_copy(table_hbm.at[idx_v.at[pl.ds(0, half)]], rows0_v, gsem0)
        g1 = pltpu.async_copy(table_hbm.at[idx_v.at[pl.ds(half, half)]], rows1_v, gsem1)
        g0.wait()
        o0 = pltpu.async_copy(rows0_v, out_hbm.at[pl.ds(base, half)], osem0)
        g1.wait()
        o1 = pltpu.async_copy(rows1_v, out_hbm.at[pl.ds(base + half, half)], osem1)
        o0.wait()
        o1.wait()

    return k(codebook, labels)


def kernel(input, codebook):
    x2 = jnp.sum(input * input, axis=1, keepdims=True)
    c2 = jnp.sum(codebook * codebook, axis=1)[None, :]
    distances, labels2d = _distances_and_labels(input, codebook, x2, c2)
    labels = labels2d.reshape(N)
    preds = _gather_preds(codebook, labels)
    return (preds, labels, distances)


# speedup vs baseline: 1.3590x; 1.3590x over previous
import functools

import jax
import jax.numpy as jnp
from jax import lax
from jax.experimental import pallas as pl
from jax.experimental.pallas import tpu as pltpu
from jax.experimental.pallas import tpu_sc as plsc

N, D, K = 8192, 256, 8192
BN = 512


def _dist_body(x_ref, c_ref, x2_ref, c2_ref, d_ref, lbl_ref):
    x = x_ref[...]
    c = c_ref[...]

    mm = lax.dot_general(
        x, c, dimension_numbers=(((1,), (1,)), ((), ())),
        preferred_element_type=jnp.float32,
    )
    d = (x2_ref[...] + c2_ref[...]) - 2.0 * mm
    d_ref[...] = d

    lmin = jnp.min(d, axis=1, keepdims=True)
    col = lax.broadcasted_iota(jnp.int32, (1, K), 1).astype(jnp.float32)
    larg = jnp.min(jnp.where(d == lmin, col, 2.0 * K), axis=1, keepdims=True)
    lbl_ref[...] = larg.astype(jnp.int32)


def _distances_and_labels(x, c, x2, c2):
    return pl.pallas_call(
        _dist_body,
        grid=(N // BN,),
        in_specs=[
            pl.BlockSpec((BN, D), lambda i: (i, 0)),
            pl.BlockSpec((K, D), lambda i: (0, 0)),
            pl.BlockSpec((BN, 1), lambda i: (i, 0)),
            pl.BlockSpec((1, K), lambda i: (0, 0)),
        ],
        out_specs=[
            pl.BlockSpec((BN, K), lambda i: (i, 0)),
            pl.BlockSpec((BN, 1), lambda i: (i, 0)),
        ],
        out_shape=[
            jax.ShapeDtypeStruct((N, K), jnp.float32),
            jax.ShapeDtypeStruct((N, 1), jnp.int32),
        ],
        compiler_params=pltpu.CompilerParams(
            dimension_semantics=("arbitrary",),
        ),
    )(x, c, x2, c2)


def _gather_preds(codebook, labels):
    info = plsc.get_sparse_core_info()
    nw = info.num_cores * info.num_subcores
    b_per_w = N // nw
    mesh = plsc.VectorSubcoreMesh(core_axis_name="c", subcore_axis_name="s")

    half = b_per_w // 2

    @functools.partial(
        pl.kernel, mesh=mesh,
        out_type=jax.ShapeDtypeStruct((N, D), jnp.float32),
        scratch_types=[
            pltpu.VMEM((b_per_w,), jnp.int32),
            pltpu.VMEM((half, D), jnp.float32),
            pltpu.VMEM((half, D), jnp.float32),
            pltpu.SemaphoreType.DMA,
            pltpu.SemaphoreType.DMA,
            pltpu.SemaphoreType.DMA,
            pltpu.SemaphoreType.DMA,
        ],
    )
    def k(table_hbm, idx_hbm, out_hbm, idx_v, rows0_v, rows1_v,
          gsem0, gsem1, osem0, osem1):
        wid = lax.axis_index("s") * info.num_cores + lax.axis_index("c")
        base = wid * b_per_w
        pltpu.sync_copy(idx_hbm.at[pl.ds(base, b_per_w)], idx_v)
        g0 = pltpu.async_copy(table_hbm.at[idx_v.at[pl.ds(0, half)]], rows0_v, gsem0)
        g1 = pltpu.async_copy(table_hbm.at[idx_v.at[pl.ds(half, half)]], rows1_v, gsem1)
        g0.wait()
        o0 = pltpu.async_copy(rows0_v, out_hbm.at[pl.ds(base, half)], osem0)
        g1.wait()
        o1 = pltpu.async_copy(rows1_v, out_hbm.at[pl.ds(base + half, half)], osem1)
        o0.wait()
        o1.wait()

    return k(codebook, labels)


def kernel(input, codebook):
    x2 = jnp.sum(input * input, axis=1, keepdims=True)
    c2 = jnp.sum(codebook * codebook, axis=1)[None, :]
    distances, labels2d = _distances_and_labels(input, codebook, x2, c2)
    labels = labels2d.reshape(N)
    preds = _gather_preds(codebook, labels)
    return (preds, labels, distances)
